# baseline (speedup 1.0000x reference)
import jax
import jax.numpy as jnp
from jax import lax
from jax.experimental import pallas as pl
from jax.experimental.pallas import tpu as pltpu
from jax.experimental.pallas import tpu_sc as plsc

NC = 2
NS = 16
MAXLEN = 200
EMBED = 64
BATCH = 4096


def _body(table_t, idx_t, pos_t, out_hbm, buf):
    wid = lax.axis_index("s") * NC + lax.axis_index("c")
    pltpu.sync_copy(table_t.at[0, pl.ds(0, 4096)], buf)

    @pl.when(wid == 0)
    def _():
        pltpu.sync_copy(buf, out_hbm.at[0, 0])


def _run(table_t, idx_t, pos_t):
    kfn = pl.kernel(
        _body,
        out_type=jax.ShapeDtypeStruct((MAXLEN, EMBED, BATCH), jnp.float32),
        mesh=plsc.VectorSubcoreMesh(
            core_axis_name="c", subcore_axis_name="s",
            num_cores=NC, num_subcores=NS),
        scratch_types=[
            pltpu.VMEM((BATCH,), jnp.float32),
        ],
        compiler_params=pltpu.CompilerParams(use_tc_tiling_on_sc=False),
    )
    return kfn(table_t, idx_t, pos_t)


def kernel(inputs, token_table, pos_table):
    out = _run(jnp.transpose(token_table), jnp.transpose(inputs),
               jnp.transpose(pos_table))
    return jnp.transpose(out, (2, 0, 1))


# speedup vs baseline: 3.5949x; 3.5949x over previous
import jax
import jax.numpy as jnp
from jax import lax
from jax.experimental import pallas as pl
from jax.experimental.pallas import tpu as pltpu
from jax.experimental.pallas import tpu_sc as plsc

NC = 2
NS = 16
NW = NC * NS

MAXLEN = 200
EMBED = 64
BATCH = 4096

SEQ_PER_W = BATCH // NW
CB0, CB1 = 128, 72
LANES = 16
NBUF = 4
NCHUNK = 2 * SEQ_PER_W

_CBS = (CB0, CB1)


def _body(idx_hbm, table_hbm, pos_hbm, out_hbm, idx_v, pos_v, bufv, sems):
    bufs = tuple(bufv.at[k] for k in range(NBUF))
    gsems = tuple(sems.at[k] for k in range(NBUF))
    ssems = tuple(sems.at[NBUF + k] for k in range(NBUF))

    wid = lax.axis_index("s") * NC + lax.axis_index("c")
    base = wid * SEQ_PER_W * MAXLEN

    pltpu.sync_copy(idx_hbm.at[pl.ds(wid * SEQ_PER_W, SEQ_PER_W)], idx_v)
    pltpu.sync_copy(pos_hbm, pos_v)

    def fire_gather(bl, h, b):
        n = _CBS[h]
        pltpu.make_async_copy(
            table_hbm.at[idx_v.at[bl, pl.ds(h * CB0, n)]],
            bufs[b].at[pl.ds(0, n)], gsems[b]).start()

    def fire_store(bl, h, b):
        n = _CBS[h]
        pltpu.make_async_copy(
            bufs[b].at[pl.ds(0, n)],
            out_hbm.at[pl.ds(base + bl * MAXLEN + h * CB0, n)],
            ssems[b]).start()

    def wait_gather(h, b):
        n = _CBS[h]
        pltpu.make_async_copy(
            table_hbm.at[idx_v.at[0, pl.ds(0, n)]],
            bufs[b].at[pl.ds(0, n)], gsems[b]).wait()

    def wait_store(h, b):
        n = _CBS[h]
        pltpu.make_async_copy(
            bufs[b].at[pl.ds(0, n)],
            out_hbm.at[pl.ds(0, n)], ssems[b]).wait()

    fire_gather(0, 0, 0)
    fire_gather(0, 1, 1)

    @pl.loop(0, SEQ_PER_W, step=2)
    def seq(bl):
        for j in range(4):
            h = j % 2
            b = j
            i = 2 * bl + j
            nb = (j + 2) % 4

            @pl.when(i + 2 < NCHUNK)
            def _():
                @pl.when(i >= 2)
                def _():
                    wait_store(h, nb)
                fire_gather(bl + 1 + (j // 2), h, nb)

            wait_gather(h, b)

            buf = bufs[b]
            rev0 = MAXLEN - 1 - h * CB0

            @pl.loop(0, _CBS[h], unroll=8)
            def row(r):
                for c in range(EMBED // LANES):
                    s = pl.ds(c * LANES, LANES)
                    plsc.addupdate(buf.at[r, s], pos_v[rev0 - r, s])

            fire_store(bl + (j // 2), h, b)

    wait_store(0, 0)
    wait_store(1, 1)
    wait_store(0, 2)
    wait_store(1, 3)


def _run(idx, table, pos):
    kfn = pl.kernel(
        _body,
        out_type=jax.ShapeDtypeStruct((BATCH * MAXLEN, EMBED), jnp.float32),
        mesh=plsc.VectorSubcoreMesh(
            core_axis_name="c", subcore_axis_name="s",
            num_cores=NC, num_subcores=NS),
        scratch_types=[
            pltpu.VMEM((SEQ_PER_W, MAXLEN), jnp.int32),
            pltpu.VMEM((MAXLEN, EMBED), jnp.float32),
            pltpu.VMEM((NBUF, CB0, EMBED), jnp.float32),
            pltpu.SemaphoreType.DMA((2 * NBUF,)),
        ],
        compiler_params=pltpu.CompilerParams(
            use_tc_tiling_on_sc=False,
            skip_device_barrier=True,
            disable_bounds_checks=True,
            disable_semaphore_checks=True,
        ),
    )
    return kfn(idx, table, pos)


def kernel(inputs, token_table, pos_table):
    out = _run(inputs.astype(jnp.int32), token_table, pos_table)
    return out.reshape(BATCH, MAXLEN, EMBED)


# speedup vs baseline: 3.6001x; 1.0014x over previous
import jax
import jax.numpy as jnp
from jax import lax
from jax.experimental import pallas as pl
from jax.experimental.pallas import tpu as pltpu
from jax.experimental.pallas import tpu_sc as plsc

NC = 2
NS = 16
NW = NC * NS

MAXLEN = 200
EMBED = 64
BATCH = 4096

SEQ_PER_W = BATCH // NW
CB0, CB1 = 128, 72
LANES = 16
NBUF = 8
NCHUNK = 2 * SEQ_PER_W

_CBS = (CB0, CB1)


def _body(idx_hbm, table_hbm, pos_hbm, out_hbm, idx_v, pos_v, bufv, sems):
    bufs = tuple(bufv.at[k] for k in range(NBUF))
    gsems = tuple(sems.at[k] for k in range(NBUF))
    ssems = tuple(sems.at[NBUF + k] for k in range(NBUF))

    wid = lax.axis_index("s") * NC + lax.axis_index("c")
    base = wid * SEQ_PER_W * MAXLEN

    pltpu.sync_copy(idx_hbm.at[pl.ds(wid * SEQ_PER_W, SEQ_PER_W)], idx_v)
    pltpu.sync_copy(pos_hbm, pos_v)

    def fire_gather(bl, h, b):
        n = _CBS[h]
        pltpu.make_async_copy(
            table_hbm.at[idx_v.at[bl, pl.ds(h * CB0, n)]],
            bufs[b].at[pl.ds(0, n)], gsems[b]).start()

    def fire_store(bl, h, b):
        n = _CBS[h]
        pltpu.make_async_copy(
            bufs[b].at[pl.ds(0, n)],
            out_hbm.at[pl.ds(base + bl * MAXLEN + h * CB0, n)],
            ssems[b]).start()

    def wait_gather(h, b):
        n = _CBS[h]
        pltpu.make_async_copy(
            table_hbm.at[idx_v.at[0, pl.ds(0, n)]],
            bufs[b].at[pl.ds(0, n)], gsems[b]).wait()

    def wait_store(h, b):
        n = _CBS[h]
        pltpu.make_async_copy(
            bufs[b].at[pl.ds(0, n)],
            out_hbm.at[pl.ds(0, n)], ssems[b]).wait()

    fire_gather(0, 0, 0)
    fire_gather(0, 1, 1)
    fire_gather(1, 0, 2)
    fire_gather(1, 1, 3)

    @pl.loop(0, SEQ_PER_W, step=4)
    def seq(bl):
        for j in range(8):
            h = j % 2
            b = j
            i = 2 * bl + j
            nb = (j + 4) % 8

            @pl.when(i + 4 < NCHUNK)
            def _():
                @pl.when(i >= 4)
                def _():
                    wait_store(h, nb)
                fire_gather(bl + 2 + (j // 2), h, nb)

            wait_gather(h, b)

            buf = bufs[b]
            rev0 = MAXLEN - 1 - h * CB0

            @pl.loop(0, _CBS[h], unroll=8)
            def row(r):
                for c in range(EMBED // LANES):
                    s = pl.ds(c * LANES, LANES)
                    plsc.addupdate(buf.at[r, s], pos_v[rev0 - r, s])

            fire_store(bl + (j // 2), h, b)

    for k in range(8):
        wait_store(k % 2, k)


def _run(idx, table, pos):
    kfn = pl.kernel(
        _body,
        out_type=jax.ShapeDtypeStruct((BATCH * MAXLEN, EMBED), jnp.float32),
        mesh=plsc.VectorSubcoreMesh(
            core_axis_name="c", subcore_axis_name="s",
            num_cores=NC, num_subcores=NS),
        scratch_types=[
            pltpu.VMEM((SEQ_PER_W, MAXLEN), jnp.int32),
            pltpu.VMEM((MAXLEN, EMBED), jnp.float32),
            pltpu.VMEM((NBUF, CB0, EMBED), jnp.float32),
            pltpu.SemaphoreType.DMA((2 * NBUF,)),
        ],
        compiler_params=pltpu.CompilerParams(use_tc_tiling_on_sc=False),
    )
    return kfn(idx, table, pos)


def kernel(inputs, token_table, pos_table):
    out = _run(inputs.astype(jnp.int32), token_table, pos_table)
    return out.reshape(BATCH, MAXLEN, EMBED)


# speedup vs baseline: 3.6036x; 1.0010x over previous
import jax
import jax.numpy as jnp
from jax import lax
from jax.experimental import pallas as pl
from jax.experimental.pallas import tpu as pltpu
from jax.experimental.pallas import tpu_sc as plsc

NC = 2
NS = 16
NW = NC * NS

MAXLEN = 200
EMBED = 64
BATCH = 4096

SEQ_PER_W = BATCH // NW
CB0, CB1 = 128, 72
LANES = 16
NBUF = 4
NCHUNK = 2 * SEQ_PER_W

_CBS = (CB0, CB1)


def _body(idx_hbm, table_hbm, pos_hbm, out_hbm, idx_v, pos_v, bufv, sems):
    bufs = tuple(bufv.at[k] for k in range(NBUF))
    gsems = tuple(sems.at[k] for k in range(NBUF))
    ssems = tuple(sems.at[NBUF + k] for k in range(NBUF))

    wid = lax.axis_index("s") * NC + lax.axis_index("c")
    base = wid * SEQ_PER_W * MAXLEN

    pltpu.sync_copy(idx_hbm.at[pl.ds(wid * SEQ_PER_W, SEQ_PER_W)], idx_v)
    pltpu.sync_copy(pos_hbm, pos_v)

    def fire_gather(bl, h, b):
        n = _CBS[h]
        pltpu.make_async_copy(
            table_hbm.at[idx_v.at[bl, pl.ds(h * CB0, n)]],
            bufs[b].at[pl.ds(0, n)], gsems[b]).start()

    def fire_store(bl, h, b):
        n = _CBS[h]
        pltpu.make_async_copy(
            bufs[b].at[pl.ds(0, n)],
            out_hbm.at[pl.ds(base + bl * MAXLEN + h * CB0, n)],
            ssems[b]).start()

    def wait_gather(h, b):
        n = _CBS[h]
        pltpu.make_async_copy(
            table_hbm.at[idx_v.at[0, pl.ds(0, n)]],
            bufs[b].at[pl.ds(0, n)], gsems[b]).wait()

    def wait_store(h, b):
        n = _CBS[h]
        pltpu.make_async_copy(
            bufs[b].at[pl.ds(0, n)],
            out_hbm.at[pl.ds(0, n)], ssems[b]).wait()

    fire_gather(0, 0, 0)
    fire_gather(0, 1, 1)

    @pl.loop(0, SEQ_PER_W, step=2)
    def seq(bl):
        for j in range(4):
            h = j % 2
            b = j
            i = 2 * bl + j
            nb = (j + 2) % 4

            @pl.when(i + 2 < NCHUNK)
            def _():
                @pl.when(i >= 2)
                def _():
                    wait_store(h, nb)
                fire_gather(bl + 1 + (j // 2), h, nb)

            wait_gather(h, b)

            buf = bufs[b]
            rev0 = MAXLEN - 1 - h * CB0

            @pl.loop(0, _CBS[h], unroll=8)
            def row(r):
                for c in range(EMBED // LANES):
                    s = pl.ds(c * LANES, LANES)
                    plsc.addupdate(buf.at[r, s], pos_v[rev0 - r, s])

            fire_store(bl + (j // 2), h, b)

    wait_store(0, 0)
    wait_store(1, 1)
    wait_store(0, 2)
    wait_store(1, 3)


def _run(idx, table, pos):
    kfn = pl.kernel(
        _body,
        out_type=jax.ShapeDtypeStruct((BATCH * MAXLEN, EMBED), jnp.float32),
        mesh=plsc.VectorSubcoreMesh(
            core_axis_name="c", subcore_axis_name="s",
            num_cores=NC, num_subcores=NS),
        scratch_types=[
            pltpu.VMEM((SEQ_PER_W, MAXLEN), jnp.int32),
            pltpu.VMEM((MAXLEN, EMBED), jnp.float32),
            pltpu.VMEM((NBUF, CB0, EMBED), jnp.float32),
            pltpu.SemaphoreType.DMA((2 * NBUF,)),
        ],
        compiler_params=pltpu.CompilerParams(use_tc_tiling_on_sc=False),
    )
    return kfn(idx, table, pos)


def kernel(inputs, token_table, pos_table):
    out = _run(inputs.astype(jnp.int32), token_table, pos_table)
    return out.reshape(BATCH, MAXLEN, EMBED)
